# baseline (speedup 1.0000x reference)
import jax
import jax.numpy as jnp
from jax import lax
from jax.experimental import pallas as pl
from jax.experimental.pallas import tpu as pltpu
from jax.experimental.pallas import tpu_sc as plsc

VOCAB = 49408
EMBED = 768
NUM_POS = 77
PADDED_POS = 80
BATCH = 4096
B2 = BATCH * PADDED_POS
NW = 32
ROWS_PER_W = B2 // NW
CHUNK = 64
NCHUNK = ROWS_PER_W // CHUNK
NPAIR = NCHUNK // 2


def _sc_body(tok_hbm, idx_hbm, out_hbm, idx_v, rows0, rows1,
             gsem0, gsem1, osem0, osem1):
    c = lax.axis_index("c")
    s = lax.axis_index("s")
    wid = s * 2 + c
    base = wid * ROWS_PER_W

    bufs = (rows0, rows1)
    gsems = (gsem0, gsem1)
    osems = (osem0, osem1)

    pltpu.sync_copy(idx_hbm.at[pl.ds(base, ROWS_PER_W)], idx_v)

    def gather_start(g, b):
        pltpu.async_copy(tok_hbm.at[idx_v.at[pl.ds(g * CHUNK, CHUNK)]],
                         bufs[b], gsems[b])

    def gather_wait(g, b):
        pltpu.make_async_copy(tok_hbm.at[idx_v.at[pl.ds(g * CHUNK, CHUNK)]],
                              bufs[b], gsems[b]).wait()

    def out_start(g, b):
        pltpu.async_copy(bufs[b], out_hbm.at[pl.ds(base + g * CHUNK, CHUNK)],
                         osems[b])

    def out_drain(b):
        pltpu.make_async_copy(bufs[b], out_hbm.at[pl.ds(base, CHUNK)],
                              osems[b]).wait()

    gather_start(0, 0)

    def visit(g, b):
        bn = 1 - b
        gather_wait(g, b)

        @pl.when(g >= 1)
        def _():
            out_drain(bn)

        @pl.when(g + 1 < NCHUNK)
        def _():
            gather_start(g + 1, bn)

        out_start(g, b)

    def pair(k2, _):
        visit(2 * k2, 0)
        visit(2 * k2 + 1, 1)
        return 0

    lax.fori_loop(0, NPAIR, pair, 0)
    out_drain(1)


def _tc_add_body(gath_ref, pos_ref, out_ref):
    out_ref[0] = gath_ref[0:NUM_POS, :] + pos_ref[...]


@jax.jit
def _run(idx, token_embedding, position_embedding):
    mesh = plsc.VectorSubcoreMesh(core_axis_name="c", subcore_axis_name="s")
    gather_k = pl.kernel(
        _sc_body,
        out_type=jax.ShapeDtypeStruct((B2, EMBED), jnp.float32),
        mesh=mesh,
        scratch_types=[
            pltpu.VMEM((ROWS_PER_W,), jnp.int32),
            pltpu.VMEM((CHUNK, EMBED), jnp.float32),
            pltpu.VMEM((CHUNK, EMBED), jnp.float32),
            pltpu.SemaphoreType.DMA,
            pltpu.SemaphoreType.DMA,
            pltpu.SemaphoreType.DMA,
            pltpu.SemaphoreType.DMA,
        ],
    )
    gath = gather_k(token_embedding, idx)

    add_k = pl.pallas_call(
        _tc_add_body,
        out_shape=jax.ShapeDtypeStruct((BATCH, NUM_POS, EMBED), jnp.float32),
        grid=(BATCH,),
        in_specs=[
            pl.BlockSpec((PADDED_POS, EMBED), lambda i: (i, 0)),
            pl.BlockSpec((NUM_POS, EMBED), lambda i: (0, 0)),
        ],
        out_specs=pl.BlockSpec((1, NUM_POS, EMBED), lambda i: (i, 0, 0)),
    )
    return add_k(gath, position_embedding)


def kernel(input_tokens, token_embedding, position_embedding):
    tokens = input_tokens.astype(jnp.int32)
    cols = jnp.minimum(jnp.arange(PADDED_POS), NUM_POS - 1)
    idx = tokens[:, cols].reshape(-1)
    return _run(idx, token_embedding, position_embedding)


# speedup vs baseline: 1.8738x; 1.8738x over previous
import jax
import jax.numpy as jnp
from jax import lax
from jax.experimental import pallas as pl
from jax.experimental.pallas import tpu as pltpu
from jax.experimental.pallas import tpu_sc as plsc

VOCAB = 49408
EMBED = 768
NUM_POS = 77
PADDED_POS = 80
BATCH = 4096
B2 = BATCH * PADDED_POS
NW = 32
ROWS_PER_W = B2 // NW
CHUNK = 64
NCHUNK = ROWS_PER_W // CHUNK
NPAIR = NCHUNK // 2


def _sc_body(tok_hbm, idx_hbm, out_hbm, idx_v, rows0, rows1,
             gsem0, gsem1, osem0, osem1):
    c = lax.axis_index("c")
    s = lax.axis_index("s")
    wid = s * 2 + c
    base = wid * ROWS_PER_W

    bufs = (rows0, rows1)
    gsems = (gsem0, gsem1)
    osems = (osem0, osem1)

    pltpu.sync_copy(idx_hbm.at[pl.ds(base, ROWS_PER_W)], idx_v)

    def gather_start(g, b):
        pltpu.async_copy(tok_hbm.at[idx_v.at[pl.ds(g * CHUNK, CHUNK)]],
                         bufs[b], gsems[b])

    def gather_wait(g, b):
        pltpu.make_async_copy(tok_hbm.at[idx_v.at[pl.ds(g * CHUNK, CHUNK)]],
                              bufs[b], gsems[b]).wait()

    def out_start(g, b):
        pltpu.async_copy(bufs[b], out_hbm.at[pl.ds(base + g * CHUNK, CHUNK)],
                         osems[b])

    def out_drain(b):
        pltpu.make_async_copy(bufs[b], out_hbm.at[pl.ds(base, CHUNK)],
                              osems[b]).wait()

    gather_start(0, 0)

    def visit(g, b):
        bn = 1 - b
        gather_wait(g, b)

        @pl.when(g >= 1)
        def _():
            out_drain(bn)

        @pl.when(g + 1 < NCHUNK)
        def _():
            gather_start(g + 1, bn)

        out_start(g, b)

    def pair(k2, _):
        visit(2 * k2, 0)
        visit(2 * k2 + 1, 1)
        return 0

    lax.fori_loop(0, NPAIR, pair, 0)
    out_drain(1)


TC_ELEMS = 8


def _tc_add_body(gath_ref, pos_ref, out_ref):
    p = pos_ref[...]
    for k in range(TC_ELEMS):
        out_ref[k] = gath_ref[k * PADDED_POS:k * PADDED_POS + NUM_POS, :] + p


@jax.jit
def _run(idx, token_embedding, position_embedding):
    mesh = plsc.VectorSubcoreMesh(core_axis_name="c", subcore_axis_name="s")
    gather_k = pl.kernel(
        _sc_body,
        out_type=jax.ShapeDtypeStruct((B2, EMBED), jnp.float32),
        mesh=mesh,
        scratch_types=[
            pltpu.VMEM((ROWS_PER_W,), jnp.int32),
            pltpu.VMEM((CHUNK, EMBED), jnp.float32),
            pltpu.VMEM((CHUNK, EMBED), jnp.float32),
            pltpu.SemaphoreType.DMA,
            pltpu.SemaphoreType.DMA,
            pltpu.SemaphoreType.DMA,
            pltpu.SemaphoreType.DMA,
        ],
    )
    gath = gather_k(token_embedding, idx)

    add_k = pl.pallas_call(
        _tc_add_body,
        out_shape=jax.ShapeDtypeStruct((BATCH, NUM_POS, EMBED), jnp.float32),
        grid=(BATCH // TC_ELEMS,),
        in_specs=[
            pl.BlockSpec((TC_ELEMS * PADDED_POS, EMBED), lambda i: (i, 0)),
            pl.BlockSpec((NUM_POS, EMBED), lambda i: (0, 0)),
        ],
        out_specs=pl.BlockSpec((TC_ELEMS, NUM_POS, EMBED),
                               lambda i: (i, 0, 0)),
    )
    return add_k(gath, position_embedding)


def kernel(input_tokens, token_embedding, position_embedding):
    tokens = input_tokens.astype(jnp.int32)
    cols = jnp.minimum(jnp.arange(PADDED_POS), NUM_POS - 1)
    idx = tokens[:, cols].reshape(-1)
    return _run(idx, token_embedding, position_embedding)


# speedup vs baseline: 1.9484x; 1.0398x over previous
import jax
import jax.numpy as jnp
from jax import lax
from jax.experimental import pallas as pl
from jax.experimental.pallas import tpu as pltpu
from jax.experimental.pallas import tpu_sc as plsc

VOCAB = 49408
EMBED = 768
NUM_POS = 77
PADDED_POS = 80
BATCH = 4096
B2 = BATCH * PADDED_POS
NW = 32
ROWS_PER_W = B2 // NW
CHUNK = 64
NCHUNK = ROWS_PER_W // CHUNK
NPAIR = NCHUNK // 2


def _sc_body(tok_hbm, idx_hbm, out_hbm, idx_v, rows0, rows1,
             gsem0, gsem1, osem0, osem1):
    c = lax.axis_index("c")
    s = lax.axis_index("s")
    wid = s * 2 + c
    base = wid * ROWS_PER_W

    bufs = (rows0, rows1)
    gsems = (gsem0, gsem1)
    osems = (osem0, osem1)

    pltpu.sync_copy(idx_hbm.at[pl.ds(base, ROWS_PER_W)], idx_v)

    def gather_start(g, b):
        pltpu.async_copy(tok_hbm.at[idx_v.at[pl.ds(g * CHUNK, CHUNK)]],
                         bufs[b], gsems[b])

    def gather_wait(g, b):
        pltpu.make_async_copy(tok_hbm.at[idx_v.at[pl.ds(g * CHUNK, CHUNK)]],
                              bufs[b], gsems[b]).wait()

    def out_start(g, b):
        pltpu.async_copy(bufs[b], out_hbm.at[pl.ds(base + g * CHUNK, CHUNK)],
                         osems[b])

    def out_drain(b):
        pltpu.make_async_copy(bufs[b], out_hbm.at[pl.ds(base, CHUNK)],
                              osems[b]).wait()

    gather_start(0, 0)

    def visit(g, b):
        bn = 1 - b
        gather_wait(g, b)

        @pl.when(g >= 1)
        def _():
            out_drain(bn)

        @pl.when(g + 1 < NCHUNK)
        def _():
            gather_start(g + 1, bn)

        out_start(g, b)

    def pair(k2, _):
        visit(2 * k2, 0)
        visit(2 * k2 + 1, 1)
        return 0

    lax.fori_loop(0, NPAIR, pair, 0)
    out_drain(1)


TC_ELEMS = 32


def _tc_add_body(gath_ref, pos_ref, out_ref):
    p = pos_ref[...]
    for k in range(TC_ELEMS):
        out_ref[k] = gath_ref[k * PADDED_POS:k * PADDED_POS + NUM_POS, :] + p


@jax.jit
def _run(idx, token_embedding, position_embedding):
    mesh = plsc.VectorSubcoreMesh(core_axis_name="c", subcore_axis_name="s")
    gather_k = pl.kernel(
        _sc_body,
        out_type=jax.ShapeDtypeStruct((B2, EMBED), jnp.float32),
        mesh=mesh,
        scratch_types=[
            pltpu.VMEM((ROWS_PER_W,), jnp.int32),
            pltpu.VMEM((CHUNK, EMBED), jnp.float32),
            pltpu.VMEM((CHUNK, EMBED), jnp.float32),
            pltpu.SemaphoreType.DMA,
            pltpu.SemaphoreType.DMA,
            pltpu.SemaphoreType.DMA,
            pltpu.SemaphoreType.DMA,
        ],
    )
    gath = gather_k(token_embedding, idx)

    add_k = pl.pallas_call(
        _tc_add_body,
        out_shape=jax.ShapeDtypeStruct((BATCH, NUM_POS, EMBED), jnp.float32),
        grid=(BATCH // TC_ELEMS,),
        in_specs=[
            pl.BlockSpec((TC_ELEMS * PADDED_POS, EMBED), lambda i: (i, 0)),
            pl.BlockSpec((NUM_POS, EMBED), lambda i: (0, 0)),
        ],
        out_specs=pl.BlockSpec((TC_ELEMS, NUM_POS, EMBED),
                               lambda i: (i, 0, 0)),
    )
    return add_k(gath, position_embedding)


def kernel(input_tokens, token_embedding, position_embedding):
    tokens = input_tokens.astype(jnp.int32)
    cols = jnp.minimum(jnp.arange(PADDED_POS), NUM_POS - 1)
    idx = tokens[:, cols].reshape(-1)
    return _run(idx, token_embedding, position_embedding)
